# baseline (speedup 1.0000x reference)
import jax
import jax.numpy as jnp
from jax import lax
from jax.experimental import pallas as pl
from jax.experimental.pallas import tpu as pltpu

_pcall = pl.pallas_call

_N_GRAPHS = 128


def _ceil_mult(n, m):
    return ((n + m - 1) // m) * m


def _pattern_kernel(adj_ij, adj_jk, adj_ik, selfcnt, p1_ref, p2_ref,
                    deg1_ref, deg2_ref, acc):
    k = pl.program_id(1)
    j = pl.program_id(2)
    nj = pl.num_programs(2)

    @pl.when(j == 0)
    def _():
        acc[...] = jnp.zeros_like(acc)

    a = adj_ij[...].astype(jnp.bfloat16)
    b = adj_jk[...].astype(jnp.bfloat16)
    acc[...] += jnp.dot(a, b, preferred_element_type=jnp.float32)

    @pl.when(j == nj - 1)
    def _():
        ti, tk = p1_ref.shape
        i0 = pl.program_id(0) * ti
        k0 = k * tk
        ri = lax.broadcasted_iota(jnp.int32, (ti, tk), 0) + i0
        ci = lax.broadcasted_iota(jnp.int32, (ti, tk), 1) + k0
        eye = ri == ci
        adjv = adj_ik[...].astype(jnp.int32) > 0
        selfdup = selfcnt[...] >= 2.0
        a1 = (adjv & jnp.logical_not(eye)) | (eye & selfdup)
        a2 = (acc[...] > 0.0) & jnp.logical_not(adjv) & jnp.logical_not(eye)
        p1_ref[...] = a1.astype(jnp.int8)
        p2_ref[...] = a2.astype(jnp.int8)
        rs1 = jnp.sum(a1.astype(jnp.float32), axis=1, keepdims=True)
        rs2 = jnp.sum(a2.astype(jnp.float32), axis=1, keepdims=True)

        @pl.when(k == 0)
        def _():
            deg1_ref[...] = rs1
            deg2_ref[...] = rs2

        @pl.when(k > 0)
        def _():
            deg1_ref[...] += rs1
            deg2_ref[...] += rs2


def _embed_kernel(x_ref, w_ref, out_ref):
    out_ref[...] = jnp.maximum(
        jnp.dot(x_ref[...], w_ref[...], preferred_element_type=jnp.float32),
        0.0)


def _hop_kernel(p1, p2, r, d1j, d2j, d1i, d2i, out_ref, acc1, acc2):
    j = pl.program_id(1)
    nj = pl.num_programs(1)

    @pl.when(j == 0)
    def _():
        acc1[...] = jnp.zeros_like(acc1)
        acc2[...] = jnp.zeros_like(acc2)

    rv = r[...]
    r1 = rv * d1j[...]
    r2 = rv * d2j[...]
    acc1[...] += jnp.dot(p1[...].astype(jnp.float32), r1,
                         preferred_element_type=jnp.float32)
    acc2[...] += jnp.dot(p2[...].astype(jnp.float32), r2,
                         preferred_element_type=jnp.float32)

    @pl.when(j == nj - 1)
    def _():
        o1 = jnp.maximum(acc1[...] * d1i[...], 0.0)
        o2 = jnp.maximum(acc2[...] * d2i[...], 0.0)
        out_ref[...] = jnp.concatenate([o1, o2], axis=1)


def _pool_kernel(h_ref, batch_ref, wout_ref, b_ref, out_ref):
    bt = batch_ref[...]
    g = out_ref.shape[0]
    gi = lax.broadcasted_iota(jnp.int32, (g, bt.shape[1]), 0)
    m = (gi == bt).astype(jnp.float32)
    sums = jnp.dot(m, h_ref[...], preferred_element_type=jnp.float32)
    cnt = jnp.sum(m, axis=1, keepdims=True)
    feat = sums / jnp.maximum(cnt, 1.0)
    out = lax.dot_general(feat, wout_ref[...], (((1,), (1,)), ((), ())),
                          preferred_element_type=jnp.float32)
    out_ref[...] = out + b_ref[...]


def kernel(x, edge_index, batch, w_embed, W_out, b_out):
    n, feat = x.shape
    hidden = w_embed.shape[1]
    g = _N_GRAPHS

    if n >= 2048:
        ti, tj = 1024, 2048
        n_pad = _ceil_mult(n, 2048)
    else:
        n_pad = _ceil_mult(n, 8)
        ti = tj = n_pad

    row = edge_index[0]
    col = edge_index[1]
    adj = jnp.zeros((n_pad, n_pad), jnp.int8).at[row, col].set(1)
    selfcnt = jnp.zeros((n_pad,), jnp.float32).at[row].add(
        (row == col).astype(jnp.float32)).reshape(n_pad, 1)

    gi_, gk_, gj_ = n_pad // ti, n_pad // ti, n_pad // tj
    p1, p2, deg1, deg2 = _pcall(
        _pattern_kernel,
        grid=(gi_, gk_, gj_),
        in_specs=[
            pl.BlockSpec((ti, tj), lambda i, k, j: (i, j)),
            pl.BlockSpec((tj, ti), lambda i, k, j: (j, k)),
            pl.BlockSpec((ti, ti), lambda i, k, j: (i, k)),
            pl.BlockSpec((ti, 1), lambda i, k, j: (i, 0)),
        ],
        out_specs=[
            pl.BlockSpec((ti, ti), lambda i, k, j: (i, k)),
            pl.BlockSpec((ti, ti), lambda i, k, j: (i, k)),
            pl.BlockSpec((ti, 1), lambda i, k, j: (i, 0)),
            pl.BlockSpec((ti, 1), lambda i, k, j: (i, 0)),
        ],
        out_shape=[
            jax.ShapeDtypeStruct((n_pad, n_pad), jnp.int8),
            jax.ShapeDtypeStruct((n_pad, n_pad), jnp.int8),
            jax.ShapeDtypeStruct((n_pad, 1), jnp.float32),
            jax.ShapeDtypeStruct((n_pad, 1), jnp.float32),
        ],
        scratch_shapes=[pltpu.VMEM((ti, ti), jnp.float32)],
    )(adj, adj, adj, selfcnt)

    dinv1 = jnp.where(deg1 > 0, deg1 ** -0.5, 0.0)
    dinv2 = jnp.where(deg2 > 0, deg2 ** -0.5, 0.0)

    x_pad = jnp.pad(x, ((0, n_pad - n), (0, 0)))
    r0 = _pcall(
        _embed_kernel,
        grid=(n_pad // ti,),
        in_specs=[
            pl.BlockSpec((ti, feat), lambda i: (i, 0)),
            pl.BlockSpec((feat, hidden), lambda i: (0, 0)),
        ],
        out_specs=pl.BlockSpec((ti, hidden), lambda i: (i, 0)),
        out_shape=jax.ShapeDtypeStruct((n_pad, hidden), jnp.float32),
    )(x_pad, w_embed)

    def _hop(r):
        f = r.shape[1]
        return _pcall(
            _hop_kernel,
            grid=(n_pad // ti, n_pad // tj),
            in_specs=[
                pl.BlockSpec((ti, tj), lambda i, j: (i, j)),
                pl.BlockSpec((ti, tj), lambda i, j: (i, j)),
                pl.BlockSpec((tj, f), lambda i, j: (j, 0)),
                pl.BlockSpec((tj, 1), lambda i, j: (j, 0)),
                pl.BlockSpec((tj, 1), lambda i, j: (j, 0)),
                pl.BlockSpec((ti, 1), lambda i, j: (i, 0)),
                pl.BlockSpec((ti, 1), lambda i, j: (i, 0)),
            ],
            out_specs=pl.BlockSpec((ti, 2 * f), lambda i, j: (i, 0)),
            out_shape=jax.ShapeDtypeStruct((n_pad, 2 * f), jnp.float32),
            scratch_shapes=[pltpu.VMEM((ti, f), jnp.float32),
                            pltpu.VMEM((ti, f), jnp.float32)],
        )(p1, p2, r, dinv1, dinv2, dinv1, dinv2)

    r1 = _hop(r0)
    r2 = _hop(r1)
    h_node = jnp.concatenate([r0, r1, r2], axis=1)
    c = h_node.shape[1]

    batch_pad = jnp.concatenate(
        [batch.astype(jnp.int32),
         jnp.full((n_pad - n,), g, jnp.int32)]).reshape(1, n_pad)
    out = _pcall(
        _pool_kernel,
        in_specs=[
            pl.BlockSpec((n_pad, c), lambda: (0, 0)),
            pl.BlockSpec((1, n_pad), lambda: (0, 0)),
            pl.BlockSpec((hidden, c), lambda: (0, 0)),
            pl.BlockSpec((1, hidden), lambda: (0, 0)),
        ],
        out_specs=pl.BlockSpec((g, hidden), lambda: (0, 0)),
        out_shape=jax.ShapeDtypeStruct((g, hidden), jnp.float32),
    )(h_node, batch_pad, W_out, b_out.reshape(1, hidden))

    return (out, jnp.zeros((), jnp.float32))
